# baseline (speedup 1.0000x reference)
import jax
import jax.numpy as jnp
from jax import lax
from jax.experimental import pallas as pl
from jax.experimental.pallas import tpu as pltpu
from jax.experimental.pallas import tpu_sc as plsc

_B, _N, _K = 16, 20000, 300
_THR = 0.5
_L = 16
_U = 4
_CAP = _N + _L * _U
_NEG = -jnp.inf
_BIG = 2**31 - 1


def _nms_body(s_hbm, l_hbm, t_hbm, r_hbm, b_hbm, out_hbm,
              s_v, l_v, t_v, r_v, b_v, out_v):
    nc = 2
    wid = lax.axis_index("s") * nc + lax.axis_index("c")

    @pl.when(wid < _B)
    def _():
        base = wid * _N
        pltpu.sync_copy(s_hbm.at[pl.ds(base, _N)], s_v.at[pl.ds(0, _N)])
        pltpu.sync_copy(l_hbm.at[pl.ds(base, _N)], l_v.at[pl.ds(0, _N)])
        pltpu.sync_copy(t_hbm.at[pl.ds(base, _N)], t_v.at[pl.ds(0, _N)])
        pltpu.sync_copy(r_hbm.at[pl.ds(base, _N)], r_v.at[pl.ds(0, _N)])
        pltpu.sync_copy(b_hbm.at[pl.ds(base, _N)], b_v.at[pl.ds(0, _N)])
        for j in range(_U):
            s_v[pl.ds(_N + j * _L, _L)] = jnp.full((_L,), _NEG, jnp.float32)

        def zero(i, _):
            out_v[pl.ds(i * _L, _L)] = jnp.zeros((_L,), jnp.float32)
            return 0

        lax.fori_loop(0, (_K * 4) // _L, zero, 0)

        lane = lax.iota(jnp.int32, _L)

        @plsc.parallel_loop(0, (_N + _L * _U - 1) // (_L * _U),
                            carry=(jnp.full((_L,), _NEG, jnp.float32),
                                   jnp.zeros((_L,), jnp.int32)))
        def amax(v, carry):
            mv, mi = carry
            for j in range(_U):
                x = s_v[pl.ds((v * _U + j) * _L, _L)]
                upd = x > mv
                mv = jnp.where(upd, x, mv)
                mi = jnp.where(upd, v * _U + j, mi)
            return mv, mi

        mv0, mi0 = amax
        gmax0 = jnp.max(mv0)
        idx0 = jnp.min(jnp.where(mv0 == gmax0, mi0 * _L + lane, _BIG))

        def winner_prologue(k, idx, n):
            idx = jnp.minimum(idx, _N - 1)
            alive = n > 0
            idxv = jnp.full((_L,), idx, jnp.int32)
            wl = plsc.load_gather(l_v, [idxv])
            wt = plsc.load_gather(t_v, [idxv])
            wr = plsc.load_gather(r_v, [idxv])
            wb = plsc.load_gather(b_v, [idxv])

            box = jnp.where(lane == 0, wl,
                            jnp.where(lane == 1, wt,
                                      jnp.where(lane == 2, wr, wb)))
            plsc.store_scatter(out_v, [jnp.full((_L,), 4 * k, jnp.int32) + lane],
                               box, mask=(lane < 4) & alive)
            plsc.store_scatter(s_v, [idxv],
                               jnp.full((_L,), _NEG, jnp.float32),
                               mask=lane == 0)
            return wl, wt, wr, wb

        def overlap_mask(wl, wt, wr, wb, xl, xt, xr, xb):
            ll = jnp.maximum(wl, xl)
            tt = jnp.maximum(wt, xt)
            rr = jnp.minimum(wr, xr)
            bb = jnp.minimum(wb, xb)
            area = (xr - xl) * (xb - xt)
            ov = jnp.maximum(0.0, rr - ll) * jnp.maximum(0.0, tt - bb) / area
            return ov < _THR

        def step1(k, carry):
            idx, n = carry
            nblk = (n + _L * _U - 1) >> 6
            wl, wt, wr, wb = winner_prologue(k, idx, n)

            @plsc.parallel_loop(0, nblk,
                                carry=(jnp.int32(0),
                                       jnp.full((_L,), _NEG, jnp.float32),
                                       jnp.full((_L,), _BIG, jnp.int32)))
            def sweep(v, carry):
                off, mv, mgi = carry
                for j in range(_U):
                    base = (v * _U + j) * _L
                    xs = s_v[pl.ds(base, _L)]
                    xl = l_v[pl.ds(base, _L)]
                    xt = t_v[pl.ds(base, _L)]
                    xr = r_v[pl.ds(base, _L)]
                    xb = b_v[pl.ds(base, _L)]
                    keep = (overlap_mask(wl, wt, wr, wb, xl, xt, xr, xb)
                            & (xs != _NEG))
                    plsc.store_compressed(s_v.at[pl.ds(off, _L)], xs, mask=keep)
                    plsc.store_compressed(l_v.at[pl.ds(off, _L)], xl, mask=keep)
                    plsc.store_compressed(t_v.at[pl.ds(off, _L)], xt, mask=keep)
                    plsc.store_compressed(r_v.at[pl.ds(off, _L)], xr, mask=keep)
                    plsc.store_compressed(b_v.at[pl.ds(off, _L)], xb, mask=keep)
                    pos = off + plsc.cumsum(keep.astype(jnp.int32)) - 1
                    upd = keep & (xs > mv)
                    mv = jnp.where(upd, xs, mv)
                    mgi = jnp.where(upd, pos, mgi)
                    off = off + plsc.all_reduce_population_count(keep)[0]
                return off, mv, mgi

            n_new, mv, mgi = sweep
            for j in range(_U):
                s_v[pl.ds(n_new + j * _L, _L)] = jnp.full((_L,), _NEG,
                                                          jnp.float32)
            gmax = jnp.max(mv)
            nidx = jnp.min(jnp.where(mv == gmax, mgi, _BIG))
            return nidx, n_new

        def step2(k, carry):
            idx, n = carry
            nblk = (n + _L * _U - 1) >> 6
            wl, wt, wr, wb = winner_prologue(k, idx, n)

            @plsc.parallel_loop(0, nblk,
                                carry=(jnp.full((_L,), _NEG, jnp.float32),
                                       jnp.full((_L,), _BIG, jnp.int32)))
            def sweep(v, carry):
                mv, mgi = carry
                for j in range(_U):
                    base = (v * _U + j) * _L
                    xs = s_v[pl.ds(base, _L)]
                    xl = l_v[pl.ds(base, _L)]
                    xt = t_v[pl.ds(base, _L)]
                    xr = r_v[pl.ds(base, _L)]
                    xb = b_v[pl.ds(base, _L)]
                    xs2 = jnp.where(
                        overlap_mask(wl, wt, wr, wb, xl, xt, xr, xb),
                        xs, _NEG)
                    s_v[pl.ds(base, _L)] = xs2
                    upd = xs2 > mv
                    mv = jnp.where(upd, xs2, mv)
                    mgi = jnp.where(upd, base + lane, mgi)
                return mv, mgi

            mv, mgi = sweep
            gmax = jnp.max(mv)
            nidx = jnp.min(jnp.where(mv == gmax, mgi, _BIG))
            n_new = jnp.where(gmax > _NEG, n, 0)
            return nidx, n_new

        _K1 = 128
        c = lax.fori_loop(0, _K1, step1, (idx0, jnp.int32(_N)))
        lax.fori_loop(_K1, _K, step2, c)
        pltpu.sync_copy(out_v, out_hbm.at[pl.ds(wid * (_K * 4), _K * 4)])


def kernel(input):
    cols = jnp.moveaxis(input, -1, 0).reshape(5, _B * _N)
    s, l, t, r, b = (cols[i] for i in range(5))
    mesh = plsc.VectorSubcoreMesh(core_axis_name="c", subcore_axis_name="s",
                                  num_cores=2, num_subcores=16)
    fn = pl.kernel(
        _nms_body,
        out_type=jax.ShapeDtypeStruct((_B * _K * 4,), jnp.float32),
        mesh=mesh,
        compiler_params=pltpu.CompilerParams(needs_layout_passes=False),
        scratch_types=[
            pltpu.VMEM((_CAP,), jnp.float32),
            pltpu.VMEM((_CAP,), jnp.float32),
            pltpu.VMEM((_CAP,), jnp.float32),
            pltpu.VMEM((_CAP,), jnp.float32),
            pltpu.VMEM((_CAP,), jnp.float32),
            pltpu.VMEM((_K * 4,), jnp.float32),
        ],
    )
    out = fn(s, l, t, r, b)
    return out.reshape(_B, _K, 4)


# speedup vs baseline: 2.1978x; 2.1978x over previous
import jax
import jax.numpy as jnp
from jax import lax
from jax.experimental import pallas as pl
from jax.experimental.pallas import tpu as pltpu
from jax.experimental.pallas import tpu_sc as plsc

_B, _N, _K = 16, 20000, 300
_THR = 0.5
_L = 16
_U = 4
_CAP = _N + _L * _U
_NEG = -jnp.inf
_BIG = 2**31 - 1


def _nms_body(s_hbm, l_hbm, t_hbm, r_hbm, b_hbm, out_hbm,
              s_v, l_v, t_v, r_v, b_v, out_v):
    nc = 2
    wid = lax.axis_index("s") * nc + lax.axis_index("c")

    @pl.when(wid < _B)
    def _():
        base = wid * _N
        pltpu.sync_copy(s_hbm.at[pl.ds(base, _N)], s_v.at[pl.ds(0, _N)])
        pltpu.sync_copy(l_hbm.at[pl.ds(base, _N)], l_v.at[pl.ds(0, _N)])
        pltpu.sync_copy(t_hbm.at[pl.ds(base, _N)], t_v.at[pl.ds(0, _N)])
        pltpu.sync_copy(r_hbm.at[pl.ds(base, _N)], r_v.at[pl.ds(0, _N)])
        pltpu.sync_copy(b_hbm.at[pl.ds(base, _N)], b_v.at[pl.ds(0, _N)])
        for j in range(_U):
            s_v[pl.ds(_N + j * _L, _L)] = jnp.full((_L,), _NEG, jnp.float32)

        def zero(i, _):
            out_v[pl.ds(i * _L, _L)] = jnp.zeros((_L,), jnp.float32)
            return 0

        lax.fori_loop(0, (_K * 4) // _L, zero, 0)

        lane = lax.iota(jnp.int32, _L)

        @plsc.parallel_loop(0, _N // _L, unroll=4,
                            carry=(jnp.full((_L,), _NEG, jnp.float32),
                                   jnp.zeros((_L,), jnp.int32)))
        def amax(v, carry):
            mv, mi = carry
            x = s_v[pl.ds(v * _L, _L)]
            upd = x > mv
            return jnp.where(upd, x, mv), jnp.where(upd, v, mi)

        mv0, mi0 = amax
        gmax0 = jnp.max(mv0)
        idx0 = jnp.min(jnp.where(mv0 == gmax0, mi0 * _L + lane, _BIG))

        def winner_prologue(k, idx, n):
            idx = jnp.minimum(idx, _N - 1)
            alive = n > 0
            idxv = jnp.full((_L,), idx, jnp.int32)
            wl = plsc.load_gather(l_v, [idxv])
            wt = plsc.load_gather(t_v, [idxv])
            wr = plsc.load_gather(r_v, [idxv])
            wb = plsc.load_gather(b_v, [idxv])

            box = jnp.where(lane == 0, wl,
                            jnp.where(lane == 1, wt,
                                      jnp.where(lane == 2, wr, wb)))
            plsc.store_scatter(out_v, [jnp.full((_L,), 4 * k, jnp.int32) + lane],
                               box, mask=(lane < 4) & alive)
            plsc.store_scatter(s_v, [idxv],
                               jnp.full((_L,), _NEG, jnp.float32),
                               mask=lane == 0)
            return wl, wt, wr, wb

        def overlap_mask(wl, wt, wr, wb, xl, xt, xr, xb):
            ll = jnp.maximum(wl, xl)
            tt = jnp.maximum(wt, xt)
            rr = jnp.minimum(wr, xr)
            bb = jnp.minimum(wb, xb)
            area = (xr - xl) * (xb - xt)
            ov = jnp.maximum(0.0, rr - ll) * jnp.maximum(0.0, tt - bb) / area
            return ov < _THR

        def step1(k, carry):
            idx, n = carry
            nvec = (n + _L - 1) >> 4
            wl, wt, wr, wb = winner_prologue(k, idx, n)

            @plsc.parallel_loop(0, nvec, unroll=2,
                                carry=(jnp.int32(0),
                                       jnp.full((_L,), _NEG, jnp.float32),
                                       jnp.full((_L,), _BIG, jnp.int32)))
            def sweep(v, carry):
                off, mv, mgi = carry
                base = v * _L
                xs = s_v[pl.ds(base, _L)]
                xl = l_v[pl.ds(base, _L)]
                xt = t_v[pl.ds(base, _L)]
                xr = r_v[pl.ds(base, _L)]
                xb = b_v[pl.ds(base, _L)]
                keep = (overlap_mask(wl, wt, wr, wb, xl, xt, xr, xb)
                        & (xs != _NEG))
                plsc.store_compressed(s_v.at[pl.ds(off, _L)], xs, mask=keep)
                plsc.store_compressed(l_v.at[pl.ds(off, _L)], xl, mask=keep)
                plsc.store_compressed(t_v.at[pl.ds(off, _L)], xt, mask=keep)
                plsc.store_compressed(r_v.at[pl.ds(off, _L)], xr, mask=keep)
                plsc.store_compressed(b_v.at[pl.ds(off, _L)], xb, mask=keep)
                pos = off + plsc.cumsum(keep.astype(jnp.int32)) - 1
                upd = keep & (xs > mv)
                mv = jnp.where(upd, xs, mv)
                mgi = jnp.where(upd, pos, mgi)
                off = off + plsc.all_reduce_population_count(keep)[0]
                return off, mv, mgi

            n_new, mv, mgi = sweep
            s_v[pl.ds(n_new, _L)] = jnp.full((_L,), _NEG, jnp.float32)
            gmax = jnp.max(mv)
            nidx = jnp.min(jnp.where(mv == gmax, mgi, _BIG))
            return nidx, n_new

        def step2(k, carry):
            idx, n = carry
            nvec = (n + _L - 1) >> 4
            wl, wt, wr, wb = winner_prologue(k, idx, n)

            @plsc.parallel_loop(0, nvec, unroll=4,
                                carry=(jnp.full((_L,), _NEG, jnp.float32),
                                       jnp.full((_L,), _BIG, jnp.int32)))
            def sweep(v, carry):
                mv, mgi = carry
                base = v * _L
                xs = s_v[pl.ds(base, _L)]
                xl = l_v[pl.ds(base, _L)]
                xt = t_v[pl.ds(base, _L)]
                xr = r_v[pl.ds(base, _L)]
                xb = b_v[pl.ds(base, _L)]
                xs2 = jnp.where(
                    overlap_mask(wl, wt, wr, wb, xl, xt, xr, xb),
                    xs, _NEG)
                s_v[pl.ds(base, _L)] = xs2
                upd = xs2 > mv
                mv = jnp.where(upd, xs2, mv)
                mgi = jnp.where(upd, base + lane, mgi)
                return mv, mgi

            mv, mgi = sweep
            gmax = jnp.max(mv)
            nidx = jnp.min(jnp.where(mv == gmax, mgi, _BIG))
            n_new = jnp.where(gmax > _NEG, n, 0)
            return nidx, n_new

        _K1 = 128
        c = lax.fori_loop(0, _K1, step1, (idx0, jnp.int32(_N)))
        lax.fori_loop(_K1, _K, step2, c)
        pltpu.sync_copy(out_v, out_hbm.at[pl.ds(wid * (_K * 4), _K * 4)])


def kernel(input):
    cols = jnp.moveaxis(input, -1, 0).reshape(5, _B * _N)
    s, l, t, r, b = (cols[i] for i in range(5))
    mesh = plsc.VectorSubcoreMesh(core_axis_name="c", subcore_axis_name="s",
                                  num_cores=2, num_subcores=16)
    fn = pl.kernel(
        _nms_body,
        out_type=jax.ShapeDtypeStruct((_B * _K * 4,), jnp.float32),
        mesh=mesh,
        compiler_params=pltpu.CompilerParams(needs_layout_passes=False),
        scratch_types=[
            pltpu.VMEM((_CAP,), jnp.float32),
            pltpu.VMEM((_CAP,), jnp.float32),
            pltpu.VMEM((_CAP,), jnp.float32),
            pltpu.VMEM((_CAP,), jnp.float32),
            pltpu.VMEM((_CAP,), jnp.float32),
            pltpu.VMEM((_K * 4,), jnp.float32),
        ],
    )
    out = fn(s, l, t, r, b)
    return out.reshape(_B, _K, 4)


# speedup vs baseline: 2.2166x; 1.0086x over previous
import jax
import jax.numpy as jnp
from jax import lax
from jax.experimental import pallas as pl
from jax.experimental.pallas import tpu as pltpu
from jax.experimental.pallas import tpu_sc as plsc

_B, _N, _K = 16, 20000, 300
_THR = 0.5
_L = 16
_U = 4
_CAP = _N + _L * _U
_NEG = -jnp.inf
_BIG = 2**31 - 1


def _nms_body(s_hbm, l_hbm, t_hbm, r_hbm, b_hbm, out_hbm,
              s_v, l_v, t_v, r_v, b_v, out_v):
    nc = 2
    wid = lax.axis_index("s") * nc + lax.axis_index("c")

    @pl.when(wid < _B)
    def _():
        base = wid * _N
        pltpu.sync_copy(s_hbm.at[pl.ds(base, _N)], s_v.at[pl.ds(0, _N)])
        pltpu.sync_copy(l_hbm.at[pl.ds(base, _N)], l_v.at[pl.ds(0, _N)])
        pltpu.sync_copy(t_hbm.at[pl.ds(base, _N)], t_v.at[pl.ds(0, _N)])
        pltpu.sync_copy(r_hbm.at[pl.ds(base, _N)], r_v.at[pl.ds(0, _N)])
        pltpu.sync_copy(b_hbm.at[pl.ds(base, _N)], b_v.at[pl.ds(0, _N)])
        for j in range(_U):
            s_v[pl.ds(_N + j * _L, _L)] = jnp.full((_L,), _NEG, jnp.float32)

        def zero(i, _):
            out_v[pl.ds(i * _L, _L)] = jnp.zeros((_L,), jnp.float32)
            return 0

        lax.fori_loop(0, (_K * 4) // _L, zero, 0)

        lane = lax.iota(jnp.int32, _L)

        @plsc.parallel_loop(0, _N // _L, unroll=4,
                            carry=(jnp.full((_L,), _NEG, jnp.float32),
                                   jnp.zeros((_L,), jnp.int32)))
        def amax(v, carry):
            mv, mi = carry
            x = s_v[pl.ds(v * _L, _L)]
            upd = x > mv
            return jnp.where(upd, x, mv), jnp.where(upd, v, mi)

        mv0, mi0 = amax
        gmax0 = jnp.max(mv0)
        idx0 = jnp.min(jnp.where(mv0 == gmax0, mi0 * _L + lane, _BIG))

        def winner_prologue(k, idx, n):
            idx = jnp.minimum(idx, _N - 1)
            alive = n > 0
            idxv = jnp.full((_L,), idx, jnp.int32)
            wl = plsc.load_gather(l_v, [idxv])
            wt = plsc.load_gather(t_v, [idxv])
            wr = plsc.load_gather(r_v, [idxv])
            wb = plsc.load_gather(b_v, [idxv])

            box = jnp.where(lane == 0, wl,
                            jnp.where(lane == 1, wt,
                                      jnp.where(lane == 2, wr, wb)))
            plsc.store_scatter(out_v, [jnp.full((_L,), 4 * k, jnp.int32) + lane],
                               box, mask=(lane < 4) & alive)
            plsc.store_scatter(s_v, [idxv],
                               jnp.full((_L,), _NEG, jnp.float32),
                               mask=lane == 0)
            return wl, wt, wr, wb

        def overlap_mask(wl, wt, wr, wb, xl, xt, xr, xb):
            ll = jnp.maximum(wl, xl)
            tt = jnp.maximum(wt, xt)
            rr = jnp.minimum(wr, xr)
            bb = jnp.minimum(wb, xb)
            area = (xr - xl) * (xb - xt)
            ov = jnp.maximum(0.0, rr - ll) * jnp.maximum(0.0, tt - bb) / area
            return ov < _THR

        def step1(k, carry):
            idx, n = carry
            nvec = (n + _L - 1) >> 4
            wl, wt, wr, wb = winner_prologue(k, idx, n)

            @plsc.parallel_loop(0, nvec, unroll=2,
                                carry=(jnp.int32(0),
                                       jnp.full((_L,), _NEG, jnp.float32),
                                       jnp.full((_L,), _BIG, jnp.int32)))
            def sweep(v, carry):
                off, mv, mgi = carry
                base = v * _L
                xs = s_v[pl.ds(base, _L)]
                xl = l_v[pl.ds(base, _L)]
                xt = t_v[pl.ds(base, _L)]
                xr = r_v[pl.ds(base, _L)]
                xb = b_v[pl.ds(base, _L)]
                keep = (overlap_mask(wl, wt, wr, wb, xl, xt, xr, xb)
                        & (xs != _NEG))
                plsc.store_compressed(s_v.at[pl.ds(off, _L)], xs, mask=keep)
                plsc.store_compressed(l_v.at[pl.ds(off, _L)], xl, mask=keep)
                plsc.store_compressed(t_v.at[pl.ds(off, _L)], xt, mask=keep)
                plsc.store_compressed(r_v.at[pl.ds(off, _L)], xr, mask=keep)
                plsc.store_compressed(b_v.at[pl.ds(off, _L)], xb, mask=keep)
                pos = off + plsc.cumsum(keep.astype(jnp.int32)) - 1
                upd = keep & (xs > mv)
                mv = jnp.where(upd, xs, mv)
                mgi = jnp.where(upd, pos, mgi)
                off = off + plsc.all_reduce_population_count(keep)[0]
                return off, mv, mgi

            n_new, mv, mgi = sweep
            s_v[pl.ds(n_new, _L)] = jnp.full((_L,), _NEG, jnp.float32)
            gmax = jnp.max(mv)
            nidx = jnp.min(jnp.where(mv == gmax, mgi, _BIG))
            return nidx, n_new

        def step2(k, carry):
            idx, n = carry
            nvec = (n + _L - 1) >> 4
            wl, wt, wr, wb = winner_prologue(k, idx, n)

            @plsc.parallel_loop(0, nvec, unroll=4,
                                carry=(jnp.full((_L,), _NEG, jnp.float32),
                                       jnp.full((_L,), _BIG, jnp.int32)))
            def sweep(v, carry):
                mv, mgi = carry
                base = v * _L
                xs = s_v[pl.ds(base, _L)]
                xl = l_v[pl.ds(base, _L)]
                xt = t_v[pl.ds(base, _L)]
                xr = r_v[pl.ds(base, _L)]
                xb = b_v[pl.ds(base, _L)]
                xs2 = jnp.where(
                    overlap_mask(wl, wt, wr, wb, xl, xt, xr, xb),
                    xs, _NEG)
                s_v[pl.ds(base, _L)] = xs2
                upd = xs2 > mv
                mv = jnp.where(upd, xs2, mv)
                mgi = jnp.where(upd, base + lane, mgi)
                return mv, mgi

            mv, mgi = sweep
            gmax = jnp.max(mv)
            nidx = jnp.min(jnp.where(mv == gmax, mgi, _BIG))
            n_new = jnp.where(gmax > _NEG, n, 0)
            return nidx, n_new

        _K1 = 80
        c = lax.fori_loop(0, _K1, step1, (idx0, jnp.int32(_N)))
        lax.fori_loop(_K1, _K, step2, c)
        pltpu.sync_copy(out_v, out_hbm.at[pl.ds(wid * (_K * 4), _K * 4)])


def kernel(input):
    cols = jnp.moveaxis(input, -1, 0).reshape(5, _B * _N)
    s, l, t, r, b = (cols[i] for i in range(5))
    mesh = plsc.VectorSubcoreMesh(core_axis_name="c", subcore_axis_name="s",
                                  num_cores=2, num_subcores=16)
    fn = pl.kernel(
        _nms_body,
        out_type=jax.ShapeDtypeStruct((_B * _K * 4,), jnp.float32),
        mesh=mesh,
        compiler_params=pltpu.CompilerParams(needs_layout_passes=False),
        scratch_types=[
            pltpu.VMEM((_CAP,), jnp.float32),
            pltpu.VMEM((_CAP,), jnp.float32),
            pltpu.VMEM((_CAP,), jnp.float32),
            pltpu.VMEM((_CAP,), jnp.float32),
            pltpu.VMEM((_CAP,), jnp.float32),
            pltpu.VMEM((_K * 4,), jnp.float32),
        ],
    )
    out = fn(s, l, t, r, b)
    return out.reshape(_B, _K, 4)
